# baseline (speedup 1.0000x reference)
import functools

import jax
import jax.numpy as jnp
from jax.experimental import pallas as pl
from jax.experimental.pallas import tpu as pltpu

_HEADS = 8
_NEG = -1e30
_HI = jax.lax.Precision.HIGHEST
_STREAM = jax.lax.Precision.DEFAULT


def _agg_kernel(starts_ref, x_ref, query_ref, inw_ref, inb_ref, outw_ref,
                outb_ref, linw_ref, out_ref, m_ref, den_ref, num_ref, *,
                nsteps, tile, heads, dh, ngroups):
    pid = pl.program_id(0)

    @pl.when(pid == 0)
    def _init():
        m_ref[...] = jnp.full_like(m_ref, _NEG)
        den_ref[...] = jnp.zeros_like(den_ref)
        num_ref[...] = jnp.zeros_like(num_ref)

    emb = x_ref.shape[1]
    cols = ngroups * heads
    rr = jax.lax.broadcasted_iota(jnp.int32, (heads, emb), 1) // dh
    hh = jax.lax.broadcasted_iota(jnp.int32, (heads, emb), 0)
    hsel = (rr == hh).astype(jnp.float32)
    cc = jax.lax.broadcasted_iota(jnp.int32, (heads, cols), 1) // ngroups
    h2 = jax.lax.broadcasted_iota(jnp.int32, (heads, cols), 0)
    r2 = (cc == h2).astype(jnp.float32)
    ident = (jax.lax.broadcasted_iota(jnp.int32, (emb, emb), 0) ==
             jax.lax.broadcasted_iota(jnp.int32, (emb, emb), 1)
             ).astype(jnp.float32)

    wq = inw_ref[0:emb, :]
    wk = inw_ref[emb:2 * emb, :]
    sel0 = (jax.lax.broadcasted_iota(jnp.int32, (1, 3), 1) == 0
            ).astype(jnp.float32)
    sel1 = (jax.lax.broadcasted_iota(jnp.int32, (1, 3), 1) == 1
            ).astype(jnp.float32)
    inb = inb_ref[...]
    bq_row = jax.lax.dot_general(sel0, inb, (((1,), (0,)), ((), ())),
                                 precision=_HI)
    bk_row = jax.lax.dot_general(sel1, inb, (((1,), (0,)), ((), ())),
                                 precision=_HI)

    qrow = query_ref[0]
    qp_row = jax.lax.dot_general(qrow, wq, (((1,), (1,)), ((), ())),
                                 precision=_HI) + bq_row
    qp_col = jax.lax.dot_general(ident, qp_row, (((1,), (1,)), ((), ())),
                                 precision=_HI)
    inv = 1.0 / jnp.sqrt(jnp.float32(dh))
    a_mat = jax.lax.dot_general(hsel, qp_col * wk,
                                (((1,), (0,)), ((), ())),
                                precision=_HI) * inv
    c_row = jax.lax.dot_general(qp_row * bk_row, hsel,
                                (((1,), (1,)), ((), ())),
                                precision=_HI) * inv
    afull = jax.lax.dot_general(r2, a_mat, (((0,), (0,)), ((), ())),
                                precision=_HI)
    cb = jax.lax.dot_general(c_row, r2, (((1,), (0,)), ((), ())),
                             precision=_HI)

    x = x_ref[...]
    sb = jax.lax.dot_general(x, afull, (((1,), (1,)), ((), ())),
                             precision=_STREAM) + cb

    colg = jax.lax.broadcasted_iota(jnp.int32, (1, cols), 1) % ngroups
    scol = jnp.zeros((1, cols), jnp.int32)
    ecol = jnp.zeros((1, cols), jnp.int32)
    for g in range(ngroups):
        scol = jnp.where(colg == g, starts_ref[g], scol)
        ecol = jnp.where(colg == g, starts_ref[g + 1], ecol)
    tg = jax.lax.broadcasted_iota(jnp.int32, (tile, 1), 0) + pid * tile
    ohm = (tg >= scol) & (tg < ecol)

    bigf = jnp.where(ohm, sb, _NEG)
    m_tile = jnp.max(bigf, axis=0, keepdims=True)
    m_old = m_ref[...]
    m_new = jnp.maximum(m_old, m_tile)
    alpha = jnp.exp(m_old - m_new)
    ef = jnp.exp(bigf - m_new)
    ones_row = jnp.ones((1, tile), jnp.float32)
    den_tile = jax.lax.dot_general(ones_row, ef, (((1,), (0,)), ((), ())),
                                   precision=_STREAM)
    den_ref[...] = den_ref[...] * alpha + den_tile
    m_ref[...] = m_new
    num_ref[...] = num_ref[...] * alpha + jax.lax.dot_general(
        x, ef, (((0,), (0,)), ((), ())), precision=_STREAM)

    @pl.when(pid == nsteps - 1)
    def _fin():
        wv = inw_ref[2 * emb:3 * emb, :]
        sel2 = (jax.lax.broadcasted_iota(jnp.int32, (1, 3), 1) == 2
                ).astype(jnp.float32)
        bv_row = jax.lax.dot_general(sel2, inb_ref[...],
                                     (((1,), (0,)), ((), ())),
                                     precision=_HI)
        ybar = num_ref[...] / den_ref[...]
        blocks = []
        for h in range(heads):
            yh = ybar[:, h * ngroups:(h + 1) * ngroups]
            wvh = wv[h * dh:(h + 1) * dh, :]
            blocks.append(jax.lax.dot_general(
                yh, wvh, (((0,), (1,)), ((), ())), precision=_HI))
        ctx = jnp.concatenate(blocks, axis=1) + bv_row
        ge = jax.lax.dot_general(ctx, outw_ref[...], (((1,), (1,)), ((), ())),
                                 precision=_HI) + outb_ref[...]
        e0_col = (jax.lax.broadcasted_iota(jnp.int32, (emb, 1), 0) == 0
                  ).astype(jnp.float32)
        linw_pad = jax.lax.dot_general(e0_col, linw_ref[...],
                                       (((1,), (0,)), ((), ())),
                                       precision=_HI)
        out_ref[...] = jax.lax.dot_general(ge, linw_pad,
                                           (((1,), (1,)), ((), ())),
                                           precision=_HI)


def kernel(tree_preds, query, in_proj_w, in_proj_b, out_w, out_b, lin_w,
           lin_b, group_ids):
    n, emb = tree_preds.shape
    heads = _HEADS
    dh = emb // heads
    ngroups = 16
    nout = lin_w.shape[0]
    tile = 8192
    nsteps = n // tile

    starts = jnp.searchsorted(
        group_ids.astype(jnp.int32),
        jnp.arange(ngroups + 1, dtype=jnp.int32)).astype(jnp.int32)
    inb = in_proj_b.reshape(3, emb)
    outb = out_b.reshape(1, emb)

    def full(arr):
        return pl.BlockSpec(arr.shape, lambda i, *_: (0,) * arr.ndim)

    grid_spec = pltpu.PrefetchScalarGridSpec(
        num_scalar_prefetch=1,
        grid=(nsteps,),
        in_specs=[
            pl.BlockSpec((tile, emb), lambda i, *_: (i, 0)),
            full(query), full(in_proj_w), full(inb), full(out_w),
            full(outb), full(lin_w),
        ],
        out_specs=pl.BlockSpec((ngroups, emb), lambda i, *_: (0, 0)),
        scratch_shapes=[
            pltpu.VMEM((1, ngroups * heads), jnp.float32),
            pltpu.VMEM((1, ngroups * heads), jnp.float32),
            pltpu.VMEM((emb, ngroups * heads), jnp.float32),
        ],
    )
    res = pl.pallas_call(
        functools.partial(_agg_kernel, nsteps=nsteps, tile=tile, heads=heads,
                          dh=dh, ngroups=ngroups),
        grid_spec=grid_spec,
        out_shape=jax.ShapeDtypeStruct((ngroups, emb), jnp.float32),
    )(starts, tree_preds, query, in_proj_w, inb, out_w, outb, lin_w)
    return res[:, :nout] + lin_b


# speedup vs baseline: 1.9077x; 1.9077x over previous
import functools

import jax
import jax.numpy as jnp
from jax.experimental import pallas as pl
from jax.experimental.pallas import tpu as pltpu

_HEADS = 8
_NEG = -1e30
_HI = jax.lax.Precision.HIGHEST
_STREAM = jax.lax.Precision.DEFAULT


def _agg_kernel(gids_ref, x_ref, query_ref, inw_ref, inb_ref, outw_ref,
                outb_ref, linw_ref, linb_ref, out_ref, m_ref, den_ref,
                num_ref, starts_ref, *, nsteps, tile, heads, dh, ngroups, n):
    pid = pl.program_id(0)

    @pl.when(pid == 0)
    def _init():
        m_ref[...] = jnp.full_like(m_ref, _NEG)
        den_ref[...] = jnp.zeros_like(den_ref)
        num_ref[...] = jnp.zeros_like(num_ref)
        gid2 = gids_ref[...]
        starts_ref[0] = 0
        for g in range(1, ngroups + 1):
            starts_ref[g] = jnp.sum((gid2 < g).astype(jnp.int32))

    emb = x_ref.shape[1]
    cols = ngroups * heads
    rr = jax.lax.broadcasted_iota(jnp.int32, (heads, emb), 1) // dh
    hh = jax.lax.broadcasted_iota(jnp.int32, (heads, emb), 0)
    hsel = (rr == hh).astype(jnp.float32)
    cc = jax.lax.broadcasted_iota(jnp.int32, (heads, cols), 1) // ngroups
    h2 = jax.lax.broadcasted_iota(jnp.int32, (heads, cols), 0)
    r2 = (cc == h2).astype(jnp.float32)
    ident = (jax.lax.broadcasted_iota(jnp.int32, (emb, emb), 0) ==
             jax.lax.broadcasted_iota(jnp.int32, (emb, emb), 1)
             ).astype(jnp.float32)

    wq = inw_ref[0:emb, :]
    wk = inw_ref[emb:2 * emb, :]
    sel0 = (jax.lax.broadcasted_iota(jnp.int32, (1, 3), 1) == 0
            ).astype(jnp.float32)
    sel1 = (jax.lax.broadcasted_iota(jnp.int32, (1, 3), 1) == 1
            ).astype(jnp.float32)
    inb = inb_ref[...]
    bq_row = jax.lax.dot_general(sel0, inb, (((1,), (0,)), ((), ())),
                                 precision=_HI)
    bk_row = jax.lax.dot_general(sel1, inb, (((1,), (0,)), ((), ())),
                                 precision=_HI)

    qrow = query_ref[0]
    qp_row = jax.lax.dot_general(qrow, wq, (((1,), (1,)), ((), ())),
                                 precision=_HI) + bq_row
    qp_col = jax.lax.dot_general(ident, qp_row, (((1,), (1,)), ((), ())),
                                 precision=_HI)
    inv = 1.0 / jnp.sqrt(jnp.float32(dh))
    a_mat = jax.lax.dot_general(hsel, qp_col * wk,
                                (((1,), (0,)), ((), ())),
                                precision=_HI) * inv
    c_row = jax.lax.dot_general(qp_row * bk_row, hsel,
                                (((1,), (1,)), ((), ())),
                                precision=_HI) * inv
    afull = jax.lax.dot_general(r2, a_mat, (((0,), (0,)), ((), ())),
                                precision=_HI)
    cb = jax.lax.dot_general(c_row, r2, (((1,), (0,)), ((), ())),
                             precision=_HI)

    x = x_ref[...]
    sb = jax.lax.dot_general(x, afull, (((1,), (1,)), ((), ())),
                             precision=_STREAM) + cb

    colg = jax.lax.broadcasted_iota(jnp.int32, (1, cols), 1) % ngroups
    scol = jnp.zeros((1, cols), jnp.int32)
    ecol = jnp.zeros((1, cols), jnp.int32)
    for g in range(ngroups):
        scol = jnp.where(colg == g, starts_ref[g], scol)
        ecol = jnp.where(colg == g, starts_ref[g + 1], ecol)
    tg = jax.lax.broadcasted_iota(jnp.int32, (tile, 1), 0) + pid * tile
    ohm = (tg >= scol) & (tg < ecol)

    bigf = jnp.where(ohm, sb, _NEG)
    m_tile = jnp.max(bigf, axis=0, keepdims=True)
    m_old = m_ref[...]
    m_new = jnp.maximum(m_old, m_tile)
    alpha = jnp.exp(m_old - m_new)
    ef = jnp.exp(bigf - m_new)
    ones_row = jnp.ones((1, tile), jnp.float32)
    den_tile = jax.lax.dot_general(ones_row, ef, (((1,), (0,)), ((), ())),
                                   precision=_STREAM)
    den_ref[...] = den_ref[...] * alpha + den_tile
    m_ref[...] = m_new
    num_ref[...] = num_ref[...] * alpha + jax.lax.dot_general(
        x, ef, (((0,), (0,)), ((), ())), precision=_STREAM)

    @pl.when(pid == nsteps - 1)
    def _fin():
        wv = inw_ref[2 * emb:3 * emb, :]
        sel2 = (jax.lax.broadcasted_iota(jnp.int32, (1, 3), 1) == 2
                ).astype(jnp.float32)
        bv_row = jax.lax.dot_general(sel2, inb_ref[...],
                                     (((1,), (0,)), ((), ())),
                                     precision=_HI)
        ybar = num_ref[...] / den_ref[...]
        blocks = []
        for h in range(heads):
            yh = ybar[:, h * ngroups:(h + 1) * ngroups]
            wvh = wv[h * dh:(h + 1) * dh, :]
            blocks.append(jax.lax.dot_general(
                yh, wvh, (((0,), (1,)), ((), ())), precision=_HI))
        ctx = jnp.concatenate(blocks, axis=1) + bv_row
        ge = jax.lax.dot_general(ctx, outw_ref[...], (((1,), (1,)), ((), ())),
                                 precision=_HI) + outb_ref[...]
        linw_col = jax.lax.dot_general(ident, linw_ref[...],
                                       (((1,), (1,)), ((), ())),
                                       precision=_HI)
        out_ref[...] = jax.lax.dot_general(
            ge, linw_col, (((1,), (0,)), ((), ())),
            precision=_HI) + linb_ref[0]


def kernel(tree_preds, query, in_proj_w, in_proj_b, out_w, out_b, lin_w,
           lin_b, group_ids):
    n, emb = tree_preds.shape
    heads = _HEADS
    dh = emb // heads
    ngroups = 16
    nout = lin_w.shape[0]
    tile = 8192
    nsteps = n // tile

    gids2 = group_ids.astype(jnp.int32).reshape(n // 128, 128)
    inb = in_proj_b.reshape(3, emb)
    outb = out_b.reshape(1, emb)

    def full(arr):
        return pl.BlockSpec(arr.shape, lambda i: (0,) * arr.ndim)

    res = pl.pallas_call(
        functools.partial(_agg_kernel, nsteps=nsteps, tile=tile, heads=heads,
                          dh=dh, ngroups=ngroups, n=n),
        grid=(nsteps,),
        in_specs=[
            full(gids2),
            pl.BlockSpec((tile, emb), lambda i: (i, 0)),
            full(query), full(in_proj_w), full(inb), full(out_w),
            full(outb), full(lin_w),
            pl.BlockSpec(memory_space=pltpu.SMEM),
        ],
        out_specs=pl.BlockSpec((ngroups, nout), lambda i: (0, 0)),
        out_shape=jax.ShapeDtypeStruct((ngroups, nout), jnp.float32),
        scratch_shapes=[
            pltpu.VMEM((1, ngroups * heads), jnp.float32),
            pltpu.VMEM((1, ngroups * heads), jnp.float32),
            pltpu.VMEM((emb, ngroups * heads), jnp.float32),
            pltpu.SMEM((ngroups + 1,), jnp.int32),
        ],
    )(gids2, tree_preds, query, in_proj_w, inb, out_w, outb, lin_w, lin_b)
    return res
